# Initial kernel scaffold; baseline (speedup 1.0000x reference)
import functools

import jax
import jax.numpy as jnp
from jax import lax
from jax.experimental import pallas as pl
from jax.experimental.pallas import tpu as pltpu
from jax.experimental.pallas import tpu_sc as plsc

N = 10000
E = 320000
D = 128
WOUT = D + 16

NC = 2
NSUB = 16
EPT = E // (NC * NSUB)
K = 80
NCHUNK = EPT // K
RPT = N // NSUB


def _prep(x, W1, b1):
    BN = 1000

    def body(x_ref, w1_ref, b1_ref, a_ref, u_ref):
        xb = x_ref[...]
        w1 = w1_ref[...]
        a_ref[...] = jnp.dot(xb, w1[:D], preferred_element_type=jnp.float32) + b1_ref[...]
        u_ref[:, :D] = jnp.dot(xb, w1[D:], preferred_element_type=jnp.float32)
        u_ref[:, D:] = xb

    return pl.pallas_call(
        body,
        grid=(N // BN,),
        in_specs=[
            pl.BlockSpec((BN, D), lambda i: (i, 0)),
            pl.BlockSpec((2 * D, D), lambda i: (0, 0)),
            pl.BlockSpec((1, D), lambda i: (0, 0)),
        ],
        out_specs=[
            pl.BlockSpec((BN, D), lambda i: (i, 0)),
            pl.BlockSpec((BN, 2 * D), lambda i: (i, 0)),
        ],
        out_shape=[
            jax.ShapeDtypeStruct((N, D), jnp.float32),
            jax.ShapeDtypeStruct((N, 2 * D), jnp.float32),
        ],
    )(x, W1, b1.reshape(1, D))


def _sc_edges(A, U, W2, src, dst):
    mesh = plsc.VectorSubcoreMesh(core_axis_name="c", subcore_axis_name="s")

    @functools.partial(
        pl.kernel,
        mesh=mesh,
        out_type=jax.ShapeDtypeStruct((NC, N, WOUT), jnp.float32),
        scratch_types=[
            pltpu.VMEM((K,), jnp.int32),
            pltpu.VMEM((K,), jnp.int32),
            pltpu.VMEM((K, D), jnp.float32),
            pltpu.VMEM((K, 2 * D), jnp.float32),
            pltpu.VMEM((K, WOUT), jnp.float32),
            pltpu.VMEM((D,), jnp.float32),
            pltpu.VMEM_SHARED((N, WOUT), jnp.float32),
            pltpu.SemaphoreType.DMA,
            pltpu.SemaphoreType.DMA,
        ],
    )
    def k(a_hbm, u_hbm, w2_hbm, src_hbm, dst_hbm, out_hbm,
          sidx, didx, av, uv, sb, w2v, acc, sem_a, sem_u):
        c = lax.axis_index("c")
        s = lax.axis_index("s")
        pltpu.sync_copy(w2_hbm, w2v)

        zero16 = jnp.zeros((16,), jnp.float32)
        onehot0 = jnp.where(lax.iota(jnp.int32, 16) == 0, 1.0, 0.0)

        def zrow(i, _):
            for j in range(WOUT // 16):
                sb[i, pl.ds(16 * j, 16)] = zero16
            return _
        lax.fori_loop(0, K, zrow, None)

        def zacc(i, _):
            pltpu.sync_copy(sb.at[pl.ds(0, 25)], acc.at[pl.ds(s * RPT + i * 25, 25)])
            return _
        lax.fori_loop(0, RPT // 25, zacc, None)
        plsc.subcore_barrier()

        ebase = (c * NSUB + s) * EPT

        def chunk(ci, _):
            base = ebase + ci * K
            pltpu.sync_copy(src_hbm.at[pl.ds(base, K)], sidx)
            pltpu.sync_copy(dst_hbm.at[pl.ds(base, K)], didx)
            ga = pltpu.async_copy(a_hbm.at[didx], av, sem_a)
            gu = pltpu.async_copy(u_hbm.at[sidx], uv, sem_u)
            ga.wait()
            gu.wait()

            def edge(e, _):
                accv = zero16
                for j in range(D // 16):
                    z = av[e, pl.ds(16 * j, 16)] + uv[e, pl.ds(16 * j, 16)]
                    ez = jnp.exp(z + z)
                    t = 1.0 - 2.0 / (ez + 1.0)
                    accv = accv + t * w2v[pl.ds(16 * j, 16)]
                exv = jnp.exp(jnp.full((16,), jnp.sum(accv), jnp.float32))
                for j in range(D // 16):
                    sb[e, pl.ds(16 * j, 16)] = exv * uv[e, pl.ds(D + 16 * j, 16)]
                sb[e, pl.ds(D, 16)] = exv * onehot0
                return _
            lax.fori_loop(0, K, edge, None)
            pltpu.sync_copy(sb, acc.at[didx], add=True)
            return _
        lax.fori_loop(0, NCHUNK, chunk, None)

        plsc.subcore_barrier()
        pltpu.sync_copy(acc.at[pl.ds(s * RPT, RPT)],
                        out_hbm.at[c].at[pl.ds(s * RPT, RPT)])

    return k(A, U, W2, src, dst)


def _final(x, ns, Wfc, bfc):
    BN = 1000

    def body(x_ref, ns_ref, wfc_ref, bfc_ref, o_ref):
        sacc = ns_ref[0] + ns_ref[1]
        denom = sacc[:, D:D + 1]
        neigh = jnp.where(denom > 0.0, sacc[:, :D] / denom, 0.0)
        wfc = wfc_ref[...]
        h = (jnp.dot(x_ref[...], wfc[:D], preferred_element_type=jnp.float32)
             + jnp.dot(neigh, wfc[D:], preferred_element_type=jnp.float32)
             + bfc_ref[...])
        o_ref[...] = jnp.maximum(h, 0.0)

    return pl.pallas_call(
        body,
        grid=(N // BN,),
        in_specs=[
            pl.BlockSpec((BN, D), lambda i: (i, 0)),
            pl.BlockSpec((NC, BN, WOUT), lambda i: (0, i, 0)),
            pl.BlockSpec((2 * D, D), lambda i: (0, 0)),
            pl.BlockSpec((1, D), lambda i: (0, 0)),
        ],
        out_specs=pl.BlockSpec((BN, D), lambda i: (i, 0)),
        out_shape=jax.ShapeDtypeStruct((N, D), jnp.float32),
    )(x, ns, Wfc, bfc.reshape(1, D))


def kernel(x, edge_index, W1, b1, W2, b2, Wfc, bfc):
    src = edge_index[0].astype(jnp.int32)
    dst = edge_index[1].astype(jnp.int32)
    A, U = _prep(x, W1, b1)
    ns = _sc_edges(A, U, W2.reshape(-1), src, dst)
    return _final(x, ns, Wfc, bfc)


# speedup vs baseline: 4.0208x; 4.0208x over previous
import functools

import jax
import jax.numpy as jnp
from jax import lax
from jax.experimental import pallas as pl
from jax.experimental.pallas import tpu as pltpu
from jax.experimental.pallas import tpu_sc as plsc

N = 10000
E = 320000
D = 128
WOUT = D + 16

NC = 2
NSUB = 16
NP = 10240
EP = 327680
EPT = EP // (NC * NSUB)
K = 64
NCHUNK = EPT // K
RPT = NP // NSUB


def _prep(xp, W1, b1):
    BN = 1024

    def body(x_ref, w1_ref, b1_ref, a_ref, u_ref):
        xb = x_ref[...]
        w1 = w1_ref[...]
        a_ref[...] = jnp.dot(xb, w1[:D], preferred_element_type=jnp.float32) + b1_ref[...]
        u_ref[:, :D] = jnp.dot(xb, w1[D:], preferred_element_type=jnp.float32)
        u_ref[:, D:] = xb

    return pl.pallas_call(
        body,
        grid=(NP // BN,),
        in_specs=[
            pl.BlockSpec((BN, D), lambda i: (i, 0)),
            pl.BlockSpec((2 * D, D), lambda i: (0, 0)),
            pl.BlockSpec((1, D), lambda i: (0, 0)),
        ],
        out_specs=[
            pl.BlockSpec((BN, D), lambda i: (i, 0)),
            pl.BlockSpec((BN, 2 * D), lambda i: (i, 0)),
        ],
        out_shape=[
            jax.ShapeDtypeStruct((NP, D), jnp.float32),
            jax.ShapeDtypeStruct((NP, 2 * D), jnp.float32),
        ],
    )(xp, W1, b1.reshape(1, D))


def _sc_edges(A, U, W2, e16):
    mesh = plsc.VectorSubcoreMesh(core_axis_name="c", subcore_axis_name="s")

    @functools.partial(
        pl.kernel,
        mesh=mesh,
        compiler_params=pltpu.CompilerParams(use_tc_tiling_on_sc=False,
                                             needs_layout_passes=False),
        out_type=jax.ShapeDtypeStruct((NC, NP, WOUT), jnp.float32),
        scratch_types=[
            pltpu.VMEM((K,), jnp.int16),
            pltpu.VMEM((K,), jnp.int16),
            pltpu.VMEM((K,), jnp.int32),
            pltpu.VMEM((K,), jnp.int32),
            pltpu.VMEM((K, D), jnp.float32),
            pltpu.VMEM((K, 2 * D), jnp.float32),
            pltpu.VMEM((K, WOUT), jnp.float32),
            pltpu.VMEM((D,), jnp.float32),
            pltpu.VMEM_SHARED((NP, WOUT), jnp.float32),
            pltpu.SemaphoreType.DMA,
            pltpu.SemaphoreType.DMA,
        ],
    )
    def k(a_hbm, u_hbm, w2_hbm, e16_hbm, out_hbm,
          s16, d16, sidx, didx, av, uv, sb, w2v, acc, sem_a, sem_u):
        c = lax.axis_index("c")
        s = lax.axis_index("s")
        pltpu.sync_copy(w2_hbm, w2v)

        zero16 = jnp.zeros((16,), jnp.float32)
        onehot0 = jnp.where(lax.iota(jnp.int32, 16) == 0, 1.0, 0.0)

        def zrow(i, _):
            for j in range(WOUT // 16):
                sb[i, pl.ds(16 * j, 16)] = zero16
            return _
        lax.fori_loop(0, K, zrow, None)

        def zacc(i, _):
            pltpu.sync_copy(sb, acc.at[pl.ds(s * RPT + i * K, K)])
            return _
        lax.fori_loop(0, RPT // K, zacc, None)
        plsc.subcore_barrier()

        ebase = (c * NSUB + s) * EPT

        def chunk(ci, _):
            base = ebase + ci * K
            pltpu.sync_copy(e16_hbm.at[0].at[pl.ds(base, K)], s16)
            pltpu.sync_copy(e16_hbm.at[1].at[pl.ds(base, K)], d16)
            for g in range(K // 32):
                for buf16, buf32 in ((s16, sidx), (d16, didx)):
                    raw = plsc.bitcast(buf16[pl.ds(32 * g, 32)], jnp.int32)
                    buf32[pl.ds(32 * g, 16)] = raw & 0xFFFF
                    buf32[pl.ds(32 * g + 16, 16)] = lax.shift_right_logical(raw, 16)
            ga = pltpu.async_copy(a_hbm.at[didx], av, sem_a)
            gu = pltpu.async_copy(u_hbm.at[sidx], uv, sem_u)
            ga.wait()
            gu.wait()

            def edge(e, _):
                accv = zero16
                for j in range(D // 16):
                    z = av[e, pl.ds(16 * j, 16)] + uv[e, pl.ds(16 * j, 16)]
                    ez = jnp.exp(z + z)
                    t = 1.0 - 2.0 / (ez + 1.0)
                    accv = accv + t * w2v[pl.ds(16 * j, 16)]
                exv = jnp.exp(jnp.full((16,), jnp.sum(accv), jnp.float32))
                for j in range(D // 16):
                    sb[e, pl.ds(16 * j, 16)] = exv * uv[e, pl.ds(D + 16 * j, 16)]
                sb[e, pl.ds(D, 16)] = exv * onehot0
                return _
            lax.fori_loop(0, K, edge, None)
            pltpu.sync_copy(sb, acc.at[didx], add=True)
            return _
        lax.fori_loop(0, NCHUNK, chunk, None)

        plsc.subcore_barrier()
        pltpu.sync_copy(acc.at[pl.ds(s * RPT, RPT)],
                        out_hbm.at[c].at[pl.ds(s * RPT, RPT)])

    return k(A, U, W2, e16)


def _final(x, ns, Wfc, bfc):
    BN = 1000

    def body(x_ref, ns_ref, wfc_ref, bfc_ref, o_ref):
        sacc = ns_ref[0] + ns_ref[1]
        denom = sacc[:, D:D + 1]
        neigh = jnp.where(denom > 0.0, sacc[:, :D] / denom, 0.0)
        wfc = wfc_ref[...]
        h = (jnp.dot(x_ref[...], wfc[:D], preferred_element_type=jnp.float32)
             + jnp.dot(neigh, wfc[D:], preferred_element_type=jnp.float32)
             + bfc_ref[...])
        o_ref[...] = jnp.maximum(h, 0.0)

    return pl.pallas_call(
        body,
        grid=(N // BN,),
        in_specs=[
            pl.BlockSpec((BN, D), lambda i: (i, 0)),
            pl.BlockSpec((NC, BN, WOUT), lambda i: (0, i, 0)),
            pl.BlockSpec((2 * D, D), lambda i: (0, 0)),
            pl.BlockSpec((1, D), lambda i: (0, 0)),
        ],
        out_specs=pl.BlockSpec((BN, D), lambda i: (i, 0)),
        out_shape=jax.ShapeDtypeStruct((N, D), jnp.float32),
    )(x, ns, Wfc, bfc.reshape(1, D))


def kernel(x, edge_index, W1, b1, W2, b2, Wfc, bfc):
    xp = jnp.zeros((NP, D), jnp.float32).at[:N].set(x)
    e16 = jnp.concatenate(
        [edge_index.astype(jnp.int16),
         jnp.full((2, EP - E), NP - 1, jnp.int16)], axis=1)
    A, U = _prep(xp, W1, b1)
    ns = _sc_edges(A, U, W2.reshape(-1), e16)
    return _final(x, ns, Wfc, bfc)


# speedup vs baseline: 8.7887x; 2.1858x over previous
import functools

import jax
import jax.numpy as jnp
from jax import lax
from jax.experimental import pallas as pl
from jax.experimental.pallas import tpu as pltpu
from jax.experimental.pallas import tpu_sc as plsc

N = 10000
E = 320000
D = 128
WOUT = D + 16

NC = 2
NSUB = 16
NP = 10240
EP = 327680
EPT = EP // (NC * NSUB)
K = 32
NCHUNK = EPT // K
RPT = NP // NSUB


def _prep(xp, W1, b1):
    BN = 1024

    def body(x_ref, w1_ref, b1_ref, a_ref, u_ref):
        xb = x_ref[...]
        w1 = w1_ref[...]
        a_ref[...] = jnp.dot(xb, w1[:D], preferred_element_type=jnp.float32) + b1_ref[...]
        u_ref[:, :D] = jnp.dot(xb, w1[D:], preferred_element_type=jnp.float32)
        u_ref[:, D:] = xb

    return pl.pallas_call(
        body,
        grid=(NP // BN,),
        in_specs=[
            pl.BlockSpec((BN, D), lambda i: (i, 0)),
            pl.BlockSpec((2 * D, D), lambda i: (0, 0)),
            pl.BlockSpec((1, D), lambda i: (0, 0)),
        ],
        out_specs=[
            pl.BlockSpec((BN, D), lambda i: (i, 0)),
            pl.BlockSpec((BN, 2 * D), lambda i: (i, 0)),
        ],
        out_shape=[
            jax.ShapeDtypeStruct((NP, D), jnp.float32),
            jax.ShapeDtypeStruct((NP, 2 * D), jnp.float32),
        ],
    )(xp, W1, b1.reshape(1, D))


def _sc_edges(A, U, W2, e16):
    mesh = plsc.VectorSubcoreMesh(core_axis_name="c", subcore_axis_name="s")

    @functools.partial(
        pl.kernel,
        mesh=mesh,
        compiler_params=pltpu.CompilerParams(use_tc_tiling_on_sc=False,
                                             needs_layout_passes=False),
        out_type=jax.ShapeDtypeStruct((NC, NP, WOUT), jnp.float32),
        scratch_types=[
            [pltpu.VMEM((K,), jnp.int16)] * 2,
            [pltpu.VMEM((K,), jnp.int16)] * 2,
            [pltpu.VMEM((K,), jnp.int32)] * 2,
            [pltpu.VMEM((K,), jnp.int32)] * 2,
            [pltpu.VMEM((K,), jnp.int32)] * 2,
            [pltpu.VMEM((K, D), jnp.float32)] * 2,
            [pltpu.VMEM((K, 2 * D), jnp.float32)] * 2,
            [pltpu.VMEM((K, WOUT), jnp.float32)] * 2,
            pltpu.VMEM((D,), jnp.float32),
            pltpu.VMEM_SHARED((NP, WOUT), jnp.float32),
            [pltpu.SemaphoreType.DMA] * 2,
            [pltpu.SemaphoreType.DMA] * 2,
            [pltpu.SemaphoreType.DMA] * 2,
        ],
    )
    def k(a_hbm, u_hbm, w2_hbm, e16_hbm, out_hbm,
          s16, d16, sidx, didx, sdix, av, uv, sb, w2v, acc, sema, semu, ssem):
        c = lax.axis_index("c")
        s = lax.axis_index("s")
        pltpu.sync_copy(w2_hbm, w2v)

        zero16 = jnp.zeros((16,), jnp.float32)
        onehot0 = jnp.where(lax.iota(jnp.int32, 16) == 0, 1.0, 0.0)
        w2r = tuple(w2v[pl.ds(16 * j, 16)] for j in range(D // 16))
        s2v = zero16
        for j in range(D // 16):
            s2v = s2v + w2r[j]
        s2s = jnp.sum(s2v)

        def zrow(i, _):
            for j in range(WOUT // 16):
                sb[0][i, pl.ds(16 * j, 16)] = zero16
            return _
        lax.fori_loop(0, K, zrow, None)

        def zacc(i, _):
            pltpu.sync_copy(sb[0], acc.at[pl.ds(s * RPT + i * K, K)])
            return _
        lax.fori_loop(0, RPT // K, zacc, None)
        plsc.subcore_barrier()

        ebase = (c * NSUB + s) * EPT

        def load_idx(ci, b):
            base = ebase + ci * K
            pltpu.sync_copy(e16_hbm.at[0].at[pl.ds(base, K)], s16[b])
            pltpu.sync_copy(e16_hbm.at[1].at[pl.ds(base, K)], d16[b])
            for g in range(K // 32):
                for b16, b32 in ((s16[b], sidx[b]), (d16[b], didx[b])):
                    raw = plsc.bitcast(b16[pl.ds(32 * g, 32)], jnp.int32)
                    b32[pl.ds(32 * g, 16)] = raw & 0xFFFF
                    b32[pl.ds(32 * g + 16, 16)] = lax.shift_right_logical(raw, 16)

        def start_gather(b):
            pltpu.async_copy(a_hbm.at[didx[b]], av[b], sema[b])
            pltpu.async_copy(u_hbm.at[sidx[b]], uv[b], semu[b])

        def compute(b):
            avb, uvb, sbb = av[b], uv[b], sb[b]

            def edge(e):
                accv = zero16
                for j in range(D // 16):
                    z = avb[e, pl.ds(16 * j, 16)] + uvb[e, pl.ds(16 * j, 16)]
                    ez = jnp.exp(z + z)
                    accv = accv + w2r[j] / (ez + 1.0)
                logit = s2s - 2.0 * jnp.sum(accv)
                exv = jnp.exp(jnp.full((16,), logit, jnp.float32))
                for j in range(D // 16):
                    sbb[e, pl.ds(16 * j, 16)] = exv * uvb[e, pl.ds(D + 16 * j, 16)]
                sbb[e, pl.ds(D, 16)] = exv * onehot0
            plsc.parallel_loop(0, K, 1, unroll=2)(edge)

        load_idx(0, 0)
        start_gather(0)

        def pair(p, _):
            for b in (0, 1):
                ci = p * 2 + b
                nb = 1 - b

                @pl.when(ci + 1 < NCHUNK)
                def _prefetch():
                    load_idx(ci + 1, nb)
                    start_gather(nb)

                pltpu.make_async_copy(a_hbm.at[didx[b]], av[b], sema[b]).wait()
                pltpu.make_async_copy(u_hbm.at[sidx[b]], uv[b], semu[b]).wait()

                @pl.when(ci >= 2)
                def _drain_scatter():
                    pltpu.make_async_copy(sb[b], acc.at[sdix[b]], ssem[b]).wait()

                compute(b)
                for g in range(K // 16):
                    sdix[b][pl.ds(16 * g, 16)] = didx[b][pl.ds(16 * g, 16)]
                pltpu.async_copy(sb[b], acc.at[sdix[b]], ssem[b], add=True)
            return _
        lax.fori_loop(0, NCHUNK // 2, pair, None)

        pltpu.make_async_copy(sb[0], acc.at[sdix[0]], ssem[0]).wait()
        pltpu.make_async_copy(sb[1], acc.at[sdix[1]], ssem[1]).wait()
        plsc.subcore_barrier()
        pltpu.sync_copy(acc.at[pl.ds(s * RPT, RPT)],
                        out_hbm.at[c].at[pl.ds(s * RPT, RPT)])

    return k(A, U, W2, e16)


def _final(x, ns, Wfc, bfc):
    BN = 1000

    def body(x_ref, ns_ref, wfc_ref, bfc_ref, o_ref):
        sacc = ns_ref[0] + ns_ref[1]
        denom = sacc[:, D:D + 1]
        neigh = jnp.where(denom > 0.0, sacc[:, :D] / denom, 0.0)
        wfc = wfc_ref[...]
        h = (jnp.dot(x_ref[...], wfc[:D], preferred_element_type=jnp.float32)
             + jnp.dot(neigh, wfc[D:], preferred_element_type=jnp.float32)
             + bfc_ref[...])
        o_ref[...] = jnp.maximum(h, 0.0)

    return pl.pallas_call(
        body,
        grid=(N // BN,),
        in_specs=[
            pl.BlockSpec((BN, D), lambda i: (i, 0)),
            pl.BlockSpec((NC, BN, WOUT), lambda i: (0, i, 0)),
            pl.BlockSpec((2 * D, D), lambda i: (0, 0)),
            pl.BlockSpec((1, D), lambda i: (0, 0)),
        ],
        out_specs=pl.BlockSpec((BN, D), lambda i: (i, 0)),
        out_shape=jax.ShapeDtypeStruct((N, D), jnp.float32),
    )(x, ns, Wfc, bfc.reshape(1, D))


def kernel(x, edge_index, W1, b1, W2, b2, Wfc, bfc):
    xp = jnp.zeros((NP, D), jnp.float32).at[:N].set(x)
    e16 = jnp.concatenate(
        [edge_index.astype(jnp.int16),
         jnp.full((2, EP - E), NP - 1, jnp.int16)], axis=1)
    A, U = _prep(xp, W1, b1)
    ns = _sc_edges(A, U, W2.reshape(-1), e16)
    return _final(x, ns, Wfc, bfc)
